# baseline (speedup 1.0000x reference)
import functools

import jax
import jax.numpy as jnp
from jax import lax
from jax.experimental import pallas as pl
from jax.experimental.pallas import tpu as pltpu
from jax.experimental.pallas import tpu_sc as plsc

N = 10000
E = 320000
NG = 128
OP_CNT = 1000
SVC_CNT = 200
ST_CNT = 10

NC = 2
NS = 16
NW = NC * NS
NPAD = 10240
EDIR = 2 * E
EROWS = 5120
RPW = EROWS // NW
EPAD = EROWS * 128
NROWS = NPAD // 128
NPW = NPAD // NW
NPS = NPAD // NS

_mesh = plsc.VectorSubcoreMesh(core_axis_name="c", subcore_axis_name="s")
_sc_params = pltpu.CompilerParams(use_tc_tiling_on_sc=False)
_f32 = jnp.float32
_i32 = jnp.int32


def _zero_vmem(ref, nwords):
    z = jnp.zeros((16,), ref.dtype)

    def body(i, _):
        ref[pl.ds(i * 16, 16)] = z
        return 0

    lax.fori_loop(0, nwords // 16, body, 0)


@functools.partial(
    pl.kernel,
    out_type=(
        jax.ShapeDtypeStruct((NC, NPAD), _f32),
        jax.ShapeDtypeStruct((NG,), _f32),
    ),
    mesh=_mesh,
    compiler_params=_sc_params,
    scratch_types=[
        pltpu.VMEM((RPW, 128), _i32),
        pltpu.VMEM((128,), _f32),
        pltpu.VMEM((8, 128), _i32),
        pltpu.VMEM((1024,), _f32),
        pltpu.VMEM((NPS,), _f32),
        pltpu.VMEM_SHARED((NPAD,), _f32),
        pltpu.VMEM_SHARED((NG,), _f32),
        pltpu.SemaphoreType.DMA,
    ],
)
def _k1_deg(ends2d, gid2d, degp, cnt, endbuf, onesbuf, gidbuf, valbuf, zbuf,
            sdeg, scnt, sem):
    c = lax.axis_index("c")
    s = lax.axis_index("s")
    w = s * NC + c

    _zero_vmem(zbuf, NPS)
    pltpu.sync_copy(zbuf, sdeg.at[pl.ds(s * NPS, NPS)])

    @pl.when(jnp.logical_and(c == 0, s == 0))
    def _():
        pltpu.sync_copy(zbuf.at[pl.ds(0, NG)], scnt)

    one = jnp.full((16,), 1.0, _f32)
    for i in range(8):
        onesbuf[pl.ds(i * 16, 16)] = one

    pltpu.sync_copy(ends2d.at[pl.ds(w * RPW, RPW)], endbuf)

    plsc.subcore_barrier()

    def edge_row(j, _):
        pltpu.async_copy(onesbuf, sdeg.at[endbuf.at[j]], sem, add=True)
        return 0

    lax.fori_loop(0, RPW, edge_row, 0)

    @pl.when(jnp.logical_and(c == 0, s < 10))
    def _():
        pltpu.sync_copy(gid2d.at[pl.ds(s * 8, 8)], gidbuf)

        def mk_val(i, _):
            idx = s * 1024 + i * 16 + lax.iota(_i32, 16)
            valbuf[pl.ds(i * 16, 16)] = jnp.where(idx < N, 1.0, 0.0)
            return 0

        lax.fori_loop(0, 1024 // 16, mk_val, 0)

        def cnt_row(j, _):
            pltpu.async_copy(valbuf.at[pl.ds(j * 128, 128)],
                             scnt.at[gidbuf.at[j]], sem, add=True)
            return 0

        lax.fori_loop(0, 8, cnt_row, 0)

        def cnt_drain(j, _):
            pltpu.make_async_copy(valbuf.at[pl.ds(0, 128)],
                                  scnt.at[gidbuf.at[0]], sem).wait()
            return 0

        lax.fori_loop(0, 8, cnt_drain, 0)

    def edge_drain(j, _):
        pltpu.make_async_copy(onesbuf, sdeg.at[endbuf.at[0]], sem).wait()
        return 0

    lax.fori_loop(0, RPW, edge_drain, 0)

    plsc.subcore_barrier()

    pltpu.sync_copy(sdeg.at[pl.ds(s * NPS, NPS)],
                    degp.at[c, pl.ds(s * NPS, NPS)])

    @pl.when(jnp.logical_and(c == 0, s == 0))
    def _():
        pltpu.sync_copy(scnt, cnt)


def _k2_body(degp_ref, gid_ref, opt_ref, svct_ref, stt_ref, w1_ref,
             dis_ref, fgd_ref, opw_ref, svcw_ref, stw_ref):
    deg = degp_ref[0] + degp_ref[1]
    rid = (lax.broadcasted_iota(_i32, (NPAD // 128, 128), 0) * 128
           + lax.broadcasted_iota(_i32, (NPAD // 128, 128), 1))
    valid = jnp.logical_and(deg > 0.0, rid < N)
    dis = jnp.where(valid, lax.rsqrt(jnp.maximum(deg, 1.0)), 0.0)
    dis_ref[...] = dis
    fgd_ref[...] = gid_ref[...].astype(_f32) * 2.0 + dis
    w1 = w1_ref[...]
    opw_ref[...] = jnp.dot(opt_ref[...], w1[:32], preferred_element_type=_f32)
    svcw_ref[...] = jnp.dot(svct_ref[...], w1[32:64], preferred_element_type=_f32)
    stw_ref[...] = jnp.dot(stt_ref[...], w1[64:96], preferred_element_type=_f32)


_k2_call = pl.pallas_call(
    _k2_body,
    out_shape=(
        jax.ShapeDtypeStruct((NPAD // 128, 128), _f32),
        jax.ShapeDtypeStruct((NPAD // 128, 128), _f32),
        jax.ShapeDtypeStruct((OP_CNT, 64), _f32),
        jax.ShapeDtypeStruct((SVC_CNT, 64), _f32),
        jax.ShapeDtypeStruct((ST_CNT, 64), _f32),
    ),
)


_K5D = 8
_K5E = 32


@functools.partial(
    pl.kernel,
    out_type=(
        jax.ShapeDtypeStruct((NPAD, 64), _f32),
        jax.ShapeDtypeStruct((NC, NPAD * NG), _f32),
    ),
    mesh=_mesh,
    compiler_params=_sc_params,
    scratch_types=[
        pltpu.VMEM((NPW,), _i32),
        pltpu.VMEM((NPW,), _i32),
        pltpu.VMEM((NPW,), _i32),
        pltpu.VMEM((NPW,), _f32),
        pltpu.VMEM((NPW, 64), _f32),
        pltpu.VMEM((SVC_CNT, 64), _f32),
        pltpu.VMEM((ST_CNT, 64), _f32),
        pltpu.SemaphoreType.DMA((3,)),
        pltpu.VMEM((_K5E * 128,), _i32),
        pltpu.VMEM((_K5E * 128,), _i32),
        pltpu.VMEM((_K5D, 128), _f32),
        pltpu.VMEM((_K5D, 128), _i32),
        pltpu.VMEM((_K5D, 128), _f32),
        pltpu.VMEM((1024,), _f32),
        pltpu.VMEM_SHARED((NPAD * NG,), _f32),
        pltpu.VMEM_SHARED((NPAD,), _f32),
        pltpu.SemaphoreType.DMA((_K5D,)),
        pltpu.SemaphoreType.DMA((_K5D,)),
    ],
)
def _k35(opid, svcid, stid, dis, opw, svcw, stw, srcs, dsts, fgd,
         h1s, wgp,
         opbuf, svcbuf, stbuf, disbuf, opr, svc_t, st_t, sem_a,
         sbufq, dbufq, fgdb, flat, val, zbuf, swg, sfgd, gsem, ssem):
    c = lax.axis_index("c")
    s = lax.axis_index("s")
    w = s * NC + c
    base = w * NPW
    slab = NPAD * NG // NS

    _zero_vmem(zbuf, 1024)
    for t in range(slab // 1024):
        pltpu.sync_copy(zbuf, swg.at[pl.ds(s * slab + t * 1024, 1024)])

    @pl.when(s == 0)
    def _():
        pltpu.sync_copy(fgd, sfgd)

    pltpu.sync_copy(opid.at[pl.ds(base, NPW)], opbuf)
    pltpu.sync_copy(svcid.at[pl.ds(base, NPW)], svcbuf)
    pltpu.sync_copy(stid.at[pl.ds(base, NPW)], stbuf)
    pltpu.sync_copy(dis.at[pl.ds(base, NPW)], disbuf)
    pltpu.sync_copy(svcw, svc_t)
    pltpu.sync_copy(stw, st_t)

    descs = []
    for ci, (off, ln) in enumerate(((0, 128), (128, 128), (256, 64))):
        sl = pl.ds(off, ln)
        descs.append(pltpu.async_copy(opw.at[opbuf.at[sl]], opr.at[sl],
                                      sem_a.at[ci]))
    for d in descs:
        d.wait()

    def scale(g, _):
        dvec = disbuf[pl.ds(g * 16, 16)]
        svec = svcbuf[pl.ds(g * 16, 16)]
        tvec = stbuf[pl.ds(g * 16, 16)]
        for n in range(16):
            d = dvec[n]
            si = svec[n]
            ti = tvec[n]
            row = g * 16 + n
            for j in range(4):
                sl = pl.ds(j * 16, 16)
                opr[row, sl] = (opr[row, sl] + svc_t[si, sl]
                                + st_t[ti, sl]) * d
        return 0

    lax.fori_loop(0, NPW // 16, scale, 0)

    pltpu.sync_copy(opr, h1s.at[pl.ds(base, NPW)])

    plsc.subcore_barrier()

    def wait_scatter(p):
        pltpu.make_async_copy(val.at[p], swg.at[flat.at[p]], ssem.at[p]).wait()

    def fire_gather(p, rl):
        idx = dbufq.at[pl.ds(rl * 128, 128)]
        pltpu.async_copy(sfgd.at[idx], fgdb.at[p], gsem.at[p])

    def wait_gather(p):
        pltpu.make_async_copy(sfgd.at[dbufq.at[pl.ds(0, 128)]], fgdb.at[p],
                              gsem.at[p]).wait()

    ngrp = _K5E // _K5D
    for e in range(RPW // _K5E):
        ebase = w * RPW * 128 + e * _K5E * 128
        pltpu.sync_copy(srcs.at[pl.ds(ebase, _K5E * 128)], sbufq)
        pltpu.sync_copy(dsts.at[pl.ds(ebase, _K5E * 128)], dbufq)
        for p in range(_K5D):
            fire_gather(p, p)

        def group(g, _):
            for p in range(_K5D):
                rl = g * _K5D + p
                wait_gather(p)
                if e == 0:
                    pl.when(g > 0)(lambda p=p: wait_scatter(p))
                else:
                    wait_scatter(p)
                for k in range(8):
                    sl = pl.ds(k * 16, 16)
                    enc = fgdb[p, sl]
                    gi = enc.astype(_i32) >> 1
                    sv = sbufq[pl.ds(rl * 128 + k * 16, 16)]
                    flat[p, sl] = sv * NG + gi
                    val[p, sl] = enc - gi.astype(_f32) * 2.0
                pltpu.async_copy(val.at[p], swg.at[flat.at[p]], ssem.at[p],
                                 add=True)
                pl.when(g < ngrp - 1)(
                    lambda p=p, rl=rl: fire_gather(p, rl + _K5D))
            return 0

        lax.fori_loop(0, ngrp, group, 0)

    for p in range(_K5D):
        wait_scatter(p)

    plsc.subcore_barrier()

    pltpu.sync_copy(swg.at[pl.ds(s * slab, slab)],
                    wgp.at[c, pl.ds(s * slab, slab)])


_K4D = 8
_K4E = 80


@functools.partial(
    pl.kernel,
    out_type=jax.ShapeDtypeStruct((NC, NPAD, 64), _f32),
    mesh=_mesh,
    compiler_params=_sc_params,
    scratch_types=[
        pltpu.VMEM((_K4E * 128,), _i32),
        pltpu.VMEM((_K4E, 128), _i32),
        pltpu.VMEM((_K4D, 128, 64), _f32),
        pltpu.VMEM_SHARED((NPAD, 64), _f32),
        pltpu.SemaphoreType.DMA((_K4D,)),
        pltpu.SemaphoreType.DMA((_K4D,)),
    ],
)
def _k4_agg(srcs, dsts2d, h1s, aggp, sbuf, dbuf, gbuf, sacc, gsem, ssem):
    c = lax.axis_index("c")
    s = lax.axis_index("s")
    w = s * NC + c

    def zrow(i, _):
        for j in range(4):
            gbuf[0, i, pl.ds(j * 16, 16)] = jnp.zeros((16,), _f32)
        return 0

    lax.fori_loop(0, 128, zrow, 0)
    for t in range(NPS // 128):
        pltpu.sync_copy(gbuf.at[0], sacc.at[pl.ds(s * NPS + t * 128, 128)])

    plsc.subcore_barrier()

    def wait_scatter(p):
        pltpu.make_async_copy(gbuf.at[p], sacc.at[dbuf.at[0]],
                              ssem.at[p]).wait()

    for e in range(RPW // _K4E):
        pltpu.sync_copy(
            srcs.at[pl.ds((w * RPW + e * _K4E) * 128, _K4E * 128)], sbuf)
        pltpu.sync_copy(dsts2d.at[pl.ds(w * RPW + e * _K4E, _K4E)], dbuf)

        def group(g, _):
            base = g * _K4D
            gd = []
            for p in range(_K4D):
                pl.when(g > 0)(lambda p=p: wait_scatter(p))
                idx = sbuf.at[pl.ds((base + p) * 128, 128)]
                gd.append(pltpu.async_copy(h1s.at[idx], gbuf.at[p],
                                           gsem.at[p]))
            for p in range(_K4D):
                gd[p].wait()
                pltpu.async_copy(gbuf.at[p], sacc.at[dbuf.at[base + p]],
                                 ssem.at[p], add=True)
            return 0

        lax.fori_loop(0, _K4E // _K4D, group, 0)
        for p in range(_K4D):
            wait_scatter(p)

    plsc.subcore_barrier()

    pltpu.sync_copy(sacc.at[pl.ds(s * NPS, NPS)],
                    aggp.at[c, pl.ds(s * NPS, NPS)])


_BLK = 512
_NBLK = NPAD // _BLK


def _k6_body(aggp_ref, wgp_ref, dis_ref, w2_ref, b1_ref, b2_ref, cnt_ref,
             mu_ref, lv_ref, ge_ref):
    i = pl.program_id(0)

    @pl.when(i == 0)
    def _():
        ge_ref[...] = jnp.zeros((NG, NG), _f32)

    d = dis_ref[...]
    acc = aggp_ref[0] + aggp_ref[1]
    out1 = jnp.maximum(acc * d + b1_ref[...], 0.0)
    h2s = jnp.dot(out1, w2_ref[...], preferred_element_type=_f32) * d
    wg = wgp_ref[0] + wgp_ref[1]
    ge_ref[...] += lax.dot_general(wg, h2s, (((0,), (0,)), ((), ())),
                                   preferred_element_type=_f32)

    @pl.when(i == _NBLK - 1)
    def _():
        ge = ge_ref[...] + cnt_ref[...] * b2_ref[...]
        mu_ref[...] = ge[:, :64]
        lv_ref[...] = jnp.tanh(ge[:, 64:])


_k6_call = pl.pallas_call(
    _k6_body,
    grid=(_NBLK,),
    in_specs=[
        pl.BlockSpec((NC, _BLK, 64), lambda i: (0, i, 0)),
        pl.BlockSpec((NC, _BLK, NG), lambda i: (0, i, 0)),
        pl.BlockSpec((_BLK, 1), lambda i: (i, 0)),
        pl.BlockSpec((64, NG), lambda i: (0, 0)),
        pl.BlockSpec((1, 64), lambda i: (0, 0)),
        pl.BlockSpec((1, NG), lambda i: (0, 0)),
        pl.BlockSpec((NG, 1), lambda i: (0, 0)),
    ],
    out_specs=[
        pl.BlockSpec((NG, 64), lambda i: (0, 0)),
        pl.BlockSpec((NG, 64), lambda i: (0, 0)),
    ],
    out_shape=(
        jax.ShapeDtypeStruct((NG, 64), _f32),
        jax.ShapeDtypeStruct((NG, 64), _f32),
    ),
    scratch_shapes=[pltpu.VMEM((NG, NG), _f32)],
)


def kernel(operation_id, service_id, status_id, node_depth, edge_index,
           graph_ids, op_table, svc_table, st_table, depth_table,
           W1, b1, W2, b2):
    del node_depth, depth_table
    i32 = _i32
    e0 = edge_index[0].astype(i32)
    e1 = edge_index[1].astype(i32)
    pad_e = N + (jnp.arange(EPAD - EDIR, dtype=i32) % (NPAD - N))
    srcs = jnp.concatenate([e0, e1, pad_e])
    dsts = jnp.concatenate([e1, e0, pad_e])
    ends2d = srcs.reshape(EROWS, 128)

    pad_n = NPAD - N
    gid = jnp.concatenate([graph_ids.astype(i32), jnp.zeros((pad_n,), i32)])
    gid2d = gid.reshape(NPAD // 128, 128)
    opid = jnp.concatenate([operation_id.astype(i32), jnp.zeros((pad_n,), i32)])
    svcid = jnp.concatenate([service_id.astype(i32), jnp.zeros((pad_n,), i32)])
    stid = jnp.concatenate([status_id.astype(i32), jnp.zeros((pad_n,), i32)])

    degp, cnt = _k1_deg(ends2d, gid2d)
    dis2, fgd2, opw, svcw, stw = _k2_call(degp.reshape(NC, NPAD // 128, 128),
                                          gid2d, op_table, svc_table,
                                          st_table, W1)
    dis = dis2.reshape(NPAD)
    h1s, wgp = _k35(opid, svcid, stid, dis, opw, svcw, stw, srcs, dsts,
                    fgd2.reshape(NPAD))
    aggp = _k4_agg(srcs, dsts.reshape(EROWS, 128), h1s)

    mu, lv = _k6_call(aggp, wgp.reshape(NC, NPAD, NG), dis.reshape(NPAD, 1),
                      W2, b1.reshape(1, 64), b2.reshape(1, NG),
                      cnt.reshape(NG, 1))
    return (mu, lv)


# speedup vs baseline: 1.0143x; 1.0143x over previous
import functools

import jax
import jax.numpy as jnp
from jax import lax
from jax.experimental import pallas as pl
from jax.experimental.pallas import tpu as pltpu
from jax.experimental.pallas import tpu_sc as plsc

N = 10000
E = 320000
NG = 128
OP_CNT = 1000
SVC_CNT = 200
ST_CNT = 10

NC = 2
NS = 16
NW = NC * NS
NPAD = 10240
EDIR = 2 * E
EROWS = 5120
RPW = EROWS // NW
EPAD = EROWS * 128
NROWS = NPAD // 128
NPW = NPAD // NW
NPS = NPAD // NS

_mesh = plsc.VectorSubcoreMesh(core_axis_name="c", subcore_axis_name="s")
_sc_params = pltpu.CompilerParams(use_tc_tiling_on_sc=False)
_f32 = jnp.float32
_i32 = jnp.int32


def _zero_vmem(ref, nwords):
    z = jnp.zeros((16,), ref.dtype)

    def body(i, _):
        ref[pl.ds(i * 16, 16)] = z
        return 0

    lax.fori_loop(0, nwords // 16, body, 0)


@functools.partial(
    pl.kernel,
    out_type=(
        jax.ShapeDtypeStruct((NC, NPAD), _f32),
        jax.ShapeDtypeStruct((NG,), _f32),
    ),
    mesh=_mesh,
    compiler_params=_sc_params,
    scratch_types=[
        pltpu.VMEM((RPW, 128), _i32),
        pltpu.VMEM((128,), _f32),
        pltpu.VMEM((8, 128), _i32),
        pltpu.VMEM((1024,), _f32),
        pltpu.VMEM((NPS,), _f32),
        pltpu.VMEM_SHARED((NPAD,), _f32),
        pltpu.VMEM_SHARED((NG,), _f32),
        pltpu.SemaphoreType.DMA,
    ],
)
def _k1_deg(ends2d, gid2d, degp, cnt, endbuf, onesbuf, gidbuf, valbuf, zbuf,
            sdeg, scnt, sem):
    c = lax.axis_index("c")
    s = lax.axis_index("s")
    w = s * NC + c

    _zero_vmem(zbuf, NPS)
    pltpu.sync_copy(zbuf, sdeg.at[pl.ds(s * NPS, NPS)])

    @pl.when(jnp.logical_and(c == 0, s == 0))
    def _():
        pltpu.sync_copy(zbuf.at[pl.ds(0, NG)], scnt)

    one = jnp.full((16,), 1.0, _f32)
    for i in range(8):
        onesbuf[pl.ds(i * 16, 16)] = one

    pltpu.sync_copy(ends2d.at[pl.ds(w * RPW, RPW)], endbuf)

    plsc.subcore_barrier()

    def edge_row(j, _):
        pltpu.async_copy(onesbuf, sdeg.at[endbuf.at[j]], sem, add=True)
        return 0

    lax.fori_loop(0, RPW, edge_row, 0)

    @pl.when(jnp.logical_and(c == 0, s < 10))
    def _():
        pltpu.sync_copy(gid2d.at[pl.ds(s * 8, 8)], gidbuf)

        def mk_val(i, _):
            idx = s * 1024 + i * 16 + lax.iota(_i32, 16)
            valbuf[pl.ds(i * 16, 16)] = jnp.where(idx < N, 1.0, 0.0)
            return 0

        lax.fori_loop(0, 1024 // 16, mk_val, 0)

        def cnt_row(j, _):
            pltpu.async_copy(valbuf.at[pl.ds(j * 128, 128)],
                             scnt.at[gidbuf.at[j]], sem, add=True)
            return 0

        lax.fori_loop(0, 8, cnt_row, 0)

        def cnt_drain(j, _):
            pltpu.make_async_copy(valbuf.at[pl.ds(0, 128)],
                                  scnt.at[gidbuf.at[0]], sem).wait()
            return 0

        lax.fori_loop(0, 8, cnt_drain, 0)

    def edge_drain(j, _):
        pltpu.make_async_copy(onesbuf, sdeg.at[endbuf.at[0]], sem).wait()
        return 0

    lax.fori_loop(0, RPW, edge_drain, 0)

    plsc.subcore_barrier()

    pltpu.sync_copy(sdeg.at[pl.ds(s * NPS, NPS)],
                    degp.at[c, pl.ds(s * NPS, NPS)])

    @pl.when(jnp.logical_and(c == 0, s == 0))
    def _():
        pltpu.sync_copy(scnt, cnt)


def _k2_body(degp_ref, gid_ref, opt_ref, svct_ref, stt_ref, w1_ref,
             dis_ref, fgd_ref, opw_ref, svcw_ref, stw_ref):
    deg = degp_ref[0] + degp_ref[1]
    rid = (lax.broadcasted_iota(_i32, (NPAD // 128, 128), 0) * 128
           + lax.broadcasted_iota(_i32, (NPAD // 128, 128), 1))
    valid = jnp.logical_and(deg > 0.0, rid < N)
    dis = jnp.where(valid, lax.rsqrt(jnp.maximum(deg, 1.0)), 0.0)
    dis_ref[...] = dis
    fgd_ref[...] = gid_ref[...].astype(_f32) * 2.0 + dis
    w1 = w1_ref[...]
    opw_ref[...] = jnp.dot(opt_ref[...], w1[:32], preferred_element_type=_f32)
    svcw_ref[...] = jnp.dot(svct_ref[...], w1[32:64], preferred_element_type=_f32)
    stw_ref[...] = jnp.dot(stt_ref[...], w1[64:96], preferred_element_type=_f32)


_k2_call = pl.pallas_call(
    _k2_body,
    out_shape=(
        jax.ShapeDtypeStruct((NPAD // 128, 128), _f32),
        jax.ShapeDtypeStruct((NPAD // 128, 128), _f32),
        jax.ShapeDtypeStruct((OP_CNT, 64), _f32),
        jax.ShapeDtypeStruct((SVC_CNT, 64), _f32),
        jax.ShapeDtypeStruct((ST_CNT, 64), _f32),
    ),
)


@functools.partial(
    pl.kernel,
    out_type=jax.ShapeDtypeStruct((NPAD, 64), _f32),
    mesh=_mesh,
    compiler_params=_sc_params,
    scratch_types=[
        pltpu.VMEM((NPW,), _i32),
        pltpu.VMEM((NPW,), _i32),
        pltpu.VMEM((NPW,), _i32),
        pltpu.VMEM((NPW,), _f32),
        pltpu.VMEM((NPW, 64), _f32),
        pltpu.VMEM((SVC_CNT, 64), _f32),
        pltpu.VMEM((ST_CNT, 64), _f32),
        pltpu.SemaphoreType.DMA((3,)),
    ],
)
def _k3_h1s(opid, svcid, stid, dis, opw, svcw, stw, h1s,
            opbuf, svcbuf, stbuf, disbuf, opr, svc_t, st_t, sem_a):
    c = lax.axis_index("c")
    s = lax.axis_index("s")
    w = s * NC + c
    base = w * NPW

    pltpu.sync_copy(opid.at[pl.ds(base, NPW)], opbuf)
    pltpu.sync_copy(svcid.at[pl.ds(base, NPW)], svcbuf)
    pltpu.sync_copy(stid.at[pl.ds(base, NPW)], stbuf)
    pltpu.sync_copy(dis.at[pl.ds(base, NPW)], disbuf)
    pltpu.sync_copy(svcw, svc_t)
    pltpu.sync_copy(stw, st_t)

    descs = []
    for ci, (off, ln) in enumerate(((0, 128), (128, 128), (256, 64))):
        sl = pl.ds(off, ln)
        descs.append(pltpu.async_copy(opw.at[opbuf.at[sl]], opr.at[sl],
                                      sem_a.at[ci]))
    for d in descs:
        d.wait()

    def scale(g, _):
        dvec = disbuf[pl.ds(g * 16, 16)]
        svec = svcbuf[pl.ds(g * 16, 16)]
        tvec = stbuf[pl.ds(g * 16, 16)]
        for n in range(16):
            d = dvec[n]
            si = svec[n]
            ti = tvec[n]
            row = g * 16 + n
            for j in range(4):
                sl = pl.ds(j * 16, 16)
                opr[row, sl] = (opr[row, sl] + svc_t[si, sl]
                                + st_t[ti, sl]) * d
        return 0

    lax.fori_loop(0, NPW // 16, scale, 0)

    pltpu.sync_copy(opr, h1s.at[pl.ds(base, NPW)])


_K5D = 8
_K5E = 40


@functools.partial(
    pl.kernel,
    out_type=jax.ShapeDtypeStruct((NC, NPAD * NG), _f32),
    mesh=_mesh,
    compiler_params=_sc_params,
    scratch_types=[
        pltpu.VMEM((_K5E * 128,), _i32),
        pltpu.VMEM((_K5E * 128,), _i32),
        pltpu.VMEM((_K5D, 128), _f32),
        pltpu.VMEM((_K5D, 128), _i32),
        pltpu.VMEM((_K5D, 128), _f32),
        pltpu.VMEM((2048,), _f32),
        pltpu.VMEM_SHARED((NPAD * NG,), _f32),
        pltpu.VMEM_SHARED((NPAD,), _f32),
        pltpu.SemaphoreType.DMA((_K5D,)),
        pltpu.SemaphoreType.DMA((_K5D,)),
    ],
)
def _k5_wg(srcs, dsts, fgd, wgp, sbufq, dbufq, fgdb, flat, val,
           zbuf, swg, sfgd, gsem, ssem):
    c = lax.axis_index("c")
    s = lax.axis_index("s")
    w = s * NC + c
    slab = NPAD * NG // NS

    _zero_vmem(zbuf, 2048)
    for t in range(slab // 2048):
        pltpu.sync_copy(zbuf, swg.at[pl.ds(s * slab + t * 2048, 2048)])

    @pl.when(s == 0)
    def _():
        pltpu.sync_copy(fgd, sfgd)

    plsc.subcore_barrier()

    def wait_scatter(p):
        pltpu.make_async_copy(val.at[p], swg.at[flat.at[p]], ssem.at[p]).wait()

    def fire_gather(p, rl):
        idx = dbufq.at[pl.ds(rl * 128, 128)]
        pltpu.async_copy(sfgd.at[idx], fgdb.at[p], gsem.at[p])

    def wait_gather(p):
        pltpu.make_async_copy(sfgd.at[dbufq.at[pl.ds(0, 128)]], fgdb.at[p],
                              gsem.at[p]).wait()

    ngrp = _K5E // _K5D
    for e in range(RPW // _K5E):
        ebase = w * RPW * 128 + e * _K5E * 128
        pltpu.sync_copy(srcs.at[pl.ds(ebase, _K5E * 128)], sbufq)
        pltpu.sync_copy(dsts.at[pl.ds(ebase, _K5E * 128)], dbufq)
        for p in range(_K5D):
            fire_gather(p, p)

        def group(g, _):
            for p in range(_K5D):
                rl = g * _K5D + p
                wait_gather(p)
                if e == 0:
                    pl.when(g > 0)(lambda p=p: wait_scatter(p))
                else:
                    wait_scatter(p)
                for k in range(8):
                    sl = pl.ds(k * 16, 16)
                    enc = fgdb[p, sl]
                    gi = enc.astype(_i32) >> 1
                    sv = sbufq[pl.ds(rl * 128 + k * 16, 16)]
                    flat[p, sl] = sv * NG + gi
                    val[p, sl] = enc - gi.astype(_f32) * 2.0
                pltpu.async_copy(val.at[p], swg.at[flat.at[p]], ssem.at[p],
                                 add=True)
                pl.when(g < ngrp - 1)(
                    lambda p=p, rl=rl: fire_gather(p, rl + _K5D))
            return 0

        lax.fori_loop(0, ngrp, group, 0)

    for p in range(_K5D):
        wait_scatter(p)

    plsc.subcore_barrier()

    pltpu.sync_copy(swg.at[pl.ds(s * slab, slab)],
                    wgp.at[c, pl.ds(s * slab, slab)])


_K4D = 8
_K4E = 80


@functools.partial(
    pl.kernel,
    out_type=jax.ShapeDtypeStruct((NC, NPAD, 64), _f32),
    mesh=_mesh,
    compiler_params=_sc_params,
    scratch_types=[
        pltpu.VMEM((_K4E * 128,), _i32),
        pltpu.VMEM((_K4E, 128), _i32),
        pltpu.VMEM((_K4D, 128, 64), _f32),
        pltpu.VMEM_SHARED((NPAD, 64), _f32),
        pltpu.SemaphoreType.DMA((_K4D,)),
        pltpu.SemaphoreType.DMA((_K4D,)),
    ],
)
def _k4_agg(srcs, dsts2d, h1s, aggp, sbuf, dbuf, gbuf, sacc, gsem, ssem):
    c = lax.axis_index("c")
    s = lax.axis_index("s")
    w = s * NC + c

    def zrow(i, _):
        for j in range(4):
            gbuf[0, i, pl.ds(j * 16, 16)] = jnp.zeros((16,), _f32)
        return 0

    lax.fori_loop(0, 128, zrow, 0)
    for t in range(NPS // 128):
        pltpu.sync_copy(gbuf.at[0], sacc.at[pl.ds(s * NPS + t * 128, 128)])

    plsc.subcore_barrier()

    def wait_scatter(p):
        pltpu.make_async_copy(gbuf.at[p], sacc.at[dbuf.at[0]],
                              ssem.at[p]).wait()

    for e in range(RPW // _K4E):
        pltpu.sync_copy(
            srcs.at[pl.ds((w * RPW + e * _K4E) * 128, _K4E * 128)], sbuf)
        pltpu.sync_copy(dsts2d.at[pl.ds(w * RPW + e * _K4E, _K4E)], dbuf)

        def group(g, _):
            base = g * _K4D
            gd = []
            for p in range(_K4D):
                pl.when(g > 0)(lambda p=p: wait_scatter(p))
                idx = sbuf.at[pl.ds((base + p) * 128, 128)]
                gd.append(pltpu.async_copy(h1s.at[idx], gbuf.at[p],
                                           gsem.at[p]))
            for p in range(_K4D):
                gd[p].wait()
                pltpu.async_copy(gbuf.at[p], sacc.at[dbuf.at[base + p]],
                                 ssem.at[p], add=True)
            return 0

        lax.fori_loop(0, _K4E // _K4D, group, 0)
        for p in range(_K4D):
            wait_scatter(p)

    plsc.subcore_barrier()

    pltpu.sync_copy(sacc.at[pl.ds(s * NPS, NPS)],
                    aggp.at[c, pl.ds(s * NPS, NPS)])


_BLK = 512
_NBLK = NPAD // _BLK


def _k6_body(aggp_ref, wgp_ref, dis_ref, w2_ref, b1_ref, b2_ref, cnt_ref,
             mu_ref, lv_ref, ge_ref):
    i = pl.program_id(0)

    @pl.when(i == 0)
    def _():
        ge_ref[...] = jnp.zeros((NG, NG), _f32)

    d = dis_ref[...]
    acc = aggp_ref[0] + aggp_ref[1]
    out1 = jnp.maximum(acc * d + b1_ref[...], 0.0)
    h2s = jnp.dot(out1, w2_ref[...], preferred_element_type=_f32) * d
    wg = wgp_ref[0] + wgp_ref[1]
    ge_ref[...] += lax.dot_general(wg, h2s, (((0,), (0,)), ((), ())),
                                   preferred_element_type=_f32)

    @pl.when(i == _NBLK - 1)
    def _():
        ge = ge_ref[...] + cnt_ref[...] * b2_ref[...]
        mu_ref[...] = ge[:, :64]
        lv_ref[...] = jnp.tanh(ge[:, 64:])


_k6_call = pl.pallas_call(
    _k6_body,
    grid=(_NBLK,),
    in_specs=[
        pl.BlockSpec((NC, _BLK, 64), lambda i: (0, i, 0)),
        pl.BlockSpec((NC, _BLK, NG), lambda i: (0, i, 0)),
        pl.BlockSpec((_BLK, 1), lambda i: (i, 0)),
        pl.BlockSpec((64, NG), lambda i: (0, 0)),
        pl.BlockSpec((1, 64), lambda i: (0, 0)),
        pl.BlockSpec((1, NG), lambda i: (0, 0)),
        pl.BlockSpec((NG, 1), lambda i: (0, 0)),
    ],
    out_specs=[
        pl.BlockSpec((NG, 64), lambda i: (0, 0)),
        pl.BlockSpec((NG, 64), lambda i: (0, 0)),
    ],
    out_shape=(
        jax.ShapeDtypeStruct((NG, 64), _f32),
        jax.ShapeDtypeStruct((NG, 64), _f32),
    ),
    scratch_shapes=[pltpu.VMEM((NG, NG), _f32)],
)


def kernel(operation_id, service_id, status_id, node_depth, edge_index,
           graph_ids, op_table, svc_table, st_table, depth_table,
           W1, b1, W2, b2):
    del node_depth, depth_table
    i32 = _i32
    e0 = edge_index[0].astype(i32)
    e1 = edge_index[1].astype(i32)
    pad_e = N + (jnp.arange(EPAD - EDIR, dtype=i32) % (NPAD - N))
    srcs = jnp.concatenate([e0, e1, pad_e])
    dsts = jnp.concatenate([e1, e0, pad_e])
    ends2d = srcs.reshape(EROWS, 128)

    pad_n = NPAD - N
    gid = jnp.concatenate([graph_ids.astype(i32), jnp.zeros((pad_n,), i32)])
    gid2d = gid.reshape(NPAD // 128, 128)
    opid = jnp.concatenate([operation_id.astype(i32), jnp.zeros((pad_n,), i32)])
    svcid = jnp.concatenate([service_id.astype(i32), jnp.zeros((pad_n,), i32)])
    stid = jnp.concatenate([status_id.astype(i32), jnp.zeros((pad_n,), i32)])

    degp, cnt = _k1_deg(ends2d, gid2d)
    dis2, fgd2, opw, svcw, stw = _k2_call(degp.reshape(NC, NPAD // 128, 128),
                                          gid2d, op_table, svc_table,
                                          st_table, W1)
    dis = dis2.reshape(NPAD)
    h1s = _k3_h1s(opid, svcid, stid, dis, opw, svcw, stw)
    aggp = _k4_agg(srcs, dsts.reshape(EROWS, 128), h1s)
    wgp = _k5_wg(srcs, dsts, fgd2.reshape(NPAD))

    mu, lv = _k6_call(aggp, wgp.reshape(NC, NPAD, NG), dis.reshape(NPAD, 1),
                      W2, b1.reshape(1, 64), b2.reshape(1, NG),
                      cnt.reshape(NG, 1))
    return (mu, lv)
